# speedup vs baseline: 1.7387x; 1.7387x over previous
import functools

import jax
import jax.numpy as jnp
from jax import lax
from jax.experimental import pallas as pl
from jax.experimental.pallas import tpu as pltpu
from jax.experimental.pallas import tpu_sc as plsc

VOCAB = 1000000
HIDDEN = 128
BATCH = 4
SEQ = 4096
EPS = 1e-3

NC = 2
NS = 16
NW = NC * NS
N_TOK = BATCH * SEQ
ROWS_PER_W = N_TOK // NW
CHUNK = 128
N_CHUNKS = ROWS_PER_W // CHUNK


def _sc_gather(token_idx, word_emb):
    mesh = plsc.VectorSubcoreMesh(core_axis_name="c", subcore_axis_name="s")

    @functools.partial(
        pl.kernel,
        mesh=mesh,
        out_type=jax.ShapeDtypeStruct((NW, N_CHUNKS, CHUNK, HIDDEN), jnp.float32),
        scratch_types=[
            pltpu.VMEM((N_CHUNKS, CHUNK), jnp.int32),
            pltpu.VMEM((N_CHUNKS, CHUNK, HIDDEN), jnp.float32),
            pltpu.SemaphoreType.DMA,
        ],
    )
    def k(table_hbm, idx_hbm, out_hbm, idx_v, rows_v, sem):
        wid = lax.axis_index("s") * NC + lax.axis_index("c")
        pltpu.sync_copy(idx_hbm.at[wid], idx_v)
        copies = []
        for c in range(N_CHUNKS):
            copies.append(
                pltpu.async_copy(table_hbm.at[idx_v.at[c]], rows_v.at[c], sem)
            )
        for c in range(N_CHUNKS):
            copies[c].wait()
        pltpu.sync_copy(rows_v, out_hbm.at[wid])

    return k(word_emb, token_idx)


def _ln_body(g_ref, segf_ref, pos_ref, seg0_ref, dseg_ref, gam_ref, bet_ref, o_ref):
    x = (
        g_ref[...]
        + pos_ref[...]
        + seg0_ref[...]
        + segf_ref[...] * dseg_ref[...]
    )
    mean = jnp.mean(x, axis=-1, keepdims=True)
    xc = x - mean
    var = jnp.mean(xc * xc, axis=-1, keepdims=True)
    o_ref[...] = xc * lax.rsqrt(var + EPS) * gam_ref[...] + bet_ref[...]


def _tc_layernorm(gathered, seg_f, pos_slice, seg0, dseg, gamma, beta):
    blk = 512
    grid = (N_TOK // blk,)
    pos_blocks = SEQ // blk
    return pl.pallas_call(
        _ln_body,
        grid=grid,
        in_specs=[
            pl.BlockSpec((blk, HIDDEN), lambda i: (i, 0)),
            pl.BlockSpec((blk, 1), lambda i: (i, 0)),
            pl.BlockSpec((blk, HIDDEN), lambda i: (lax.rem(i, pos_blocks), 0)),
            pl.BlockSpec((1, HIDDEN), lambda i: (0, 0)),
            pl.BlockSpec((1, HIDDEN), lambda i: (0, 0)),
            pl.BlockSpec((1, HIDDEN), lambda i: (0, 0)),
            pl.BlockSpec((1, HIDDEN), lambda i: (0, 0)),
        ],
        out_specs=pl.BlockSpec((blk, HIDDEN), lambda i: (i, 0)),
        out_shape=jax.ShapeDtypeStruct((N_TOK, HIDDEN), jnp.float32),
    )(gathered, seg_f, pos_slice, seg0, dseg, gamma, beta)


def kernel(token, segment, word_emb, seg_emb, pos_emb, gamma, beta):
    tok = token.astype(jnp.int32).reshape(NW, N_CHUNKS, CHUNK)
    rows = _sc_gather(tok, word_emb).reshape(N_TOK, HIDDEN)
    seg_f = segment.astype(jnp.float32).reshape(N_TOK, 1)
    seg0 = seg_emb[0].reshape(1, HIDDEN)
    dseg = (seg_emb[1] - seg_emb[0]).reshape(1, HIDDEN)
    out = _tc_layernorm(
        rows,
        seg_f,
        pos_emb[:SEQ],
        seg0,
        dseg,
        gamma.reshape(1, HIDDEN),
        beta.reshape(1, HIDDEN),
    )
    return out.reshape(BATCH, SEQ, HIDDEN)
